# Initial kernel scaffold; baseline (speedup 1.0000x reference)
import jax
import jax.numpy as jnp
from jax.experimental import pallas as pl


def kernel(x, edge_index, gamma0, beta0, W1, b1, gamma1, beta1, W2_rel, b2, W2_root, gamma2, beta2, W3_rel, b3, W3_root, gamma3, beta3):
    raise NotImplementedError("write your pallas kernel here")


# speedup vs baseline: 3.1580x; 3.1580x over previous
import functools

import jax
import jax.numpy as jnp
from jax import lax
from jax.experimental import pallas as pl
from jax.experimental.pallas import tpu as pltpu
from jax.experimental.pallas import tpu_sc as plsc

N_NODES = 10000
N_EDGES = 160000
NC, NS = 2, 16
R_BLK = 2000
N_BLKS = N_NODES // R_BLK
EPS = 1e-5


AGG_C = 256
NW = NC * NS
TPT = 320
N_PAD = NW * TPT
SE = 2000
NCH = N_EDGES // SE
GE = 64
CAP = SE + 2 * GE + 16

_SC_MESH = dict(core_axis_name="c", subcore_axis_name="s")
_SC_CP = pltpu.CompilerParams(needs_layout_passes=False)


@jax.jit
def _sc_aggregate(h, src, dst, zeros_init):
    c = AGG_C

    @functools.partial(
        pl.kernel,
        out_type=jax.ShapeDtypeStruct((N_PAD, c), jnp.float32),
        mesh=plsc.VectorSubcoreMesh(**_SC_MESH),
        compiler_params=_SC_CP,
        scratch_types=[
            pltpu.VMEM((SE,), jnp.int32),
            pltpu.VMEM((SE,), jnp.int32),
            pltpu.VMEM((CAP,), jnp.int32),
            pltpu.VMEM((CAP,), jnp.int32),
            pltpu.VMEM((GE, c), jnp.float32),
            pltpu.VMEM((TPT + 8, c), jnp.float32),
        ],
    )
    def agg_kernel(h_hbm, src_hbm, dst_hbm, z_hbm, out_hbm,
                   src_ch, dst_ch, sbuf, dbuf, rbuf, acc):
        cid = lax.axis_index("c")
        sid = lax.axis_index("s")
        w = sid * NC + cid
        mybase = w * TPT
        lane = lax.iota(jnp.int32, 16)
        pltpu.sync_copy(z_hbm, acc)

        def process_group(goff):
            pltpu.sync_copy(h_hbm.at[sbuf.at[pl.ds(goff, GE)]], rbuf)

            @pl.loop(0, GE // 16)
            def _(q):
                vlad = dbuf[pl.ds(goff + q * 16, 16)]
                for r2 in range(16):
                    rowv = vlad.at[jnp.full((16,), r2, jnp.int32)].get(
                        mode="promise_in_bounds")
                    for j in range(c // 16):
                        plsc.addupdate_scatter(
                            acc, [rowv, j * 16 + lane],
                            rbuf[q * 16 + r2, pl.ds(j * 16, 16)])

        def chunk_body(ch, ptr):
            e0 = ch * SE
            pltpu.sync_copy(src_hbm.at[pl.ds(e0, SE)], src_ch)
            pltpu.sync_copy(dst_hbm.at[pl.ds(e0, SE)], dst_ch)

            def scan_body(i, ptr):
                vd = dst_ch[pl.ds(i * 16, 16)]
                vs = src_ch[pl.ds(i * 16, 16)]
                ld = vd - mybase
                m = (ld >= 0) & (ld < TPT)
                plsc.store_compressed(sbuf.at[pl.ds(ptr, 16)], vs, mask=m)
                plsc.store_compressed(dbuf.at[pl.ds(ptr, 16)], ld, mask=m)
                return ptr + jnp.max(plsc.all_reduce_population_count(m))

            ptr = lax.fori_loop(0, SE // 16, scan_body, ptr)
            ngr = ptr // GE

            def grp(gi, _):
                process_group(gi * GE)
                return 0

            lax.fori_loop(0, ngr, grp, 0)
            rem_off = ngr * GE
            for kk in range(GE // 16):
                sbuf[pl.ds(kk * 16, 16)] = sbuf[pl.ds(rem_off + kk * 16, 16)]
                dbuf[pl.ds(kk * 16, 16)] = dbuf[pl.ds(rem_off + kk * 16, 16)]
            return ptr - rem_off

        ptr = lax.fori_loop(0, NCH, chunk_body, jnp.int32(0))
        for kk in range(GE // 16):
            sbuf[pl.ds(ptr + kk * 16, 16)] = jnp.zeros((16,), jnp.int32)
            dbuf[pl.ds(ptr + kk * 16, 16)] = jnp.full((16,), TPT, jnp.int32)

        def grp_tail(gi, _):
            process_group(gi * GE)
            return 0

        lax.fori_loop(0, (ptr + GE - 1) // GE, grp_tail, 0)
        pltpu.sync_copy(acc.at[pl.ds(0, TPT)], out_hbm.at[pl.ds(mybase, TPT)])

    return agg_kernel(h, src, dst, zeros_init)


@jax.jit
def _sc_degree(src, dst, zeros_init):
    @functools.partial(
        pl.kernel,
        out_type=jax.ShapeDtypeStruct((N_PAD,), jnp.float32),
        mesh=plsc.VectorSubcoreMesh(**_SC_MESH),
        compiler_params=_SC_CP,
        scratch_types=[
            pltpu.VMEM((SE,), jnp.int32),
            pltpu.VMEM((TPT + 16,), jnp.float32),
        ],
    )
    def deg_kernel(dst_hbm, z_hbm, out_hbm, dst_ch, cnt):
        cid = lax.axis_index("c")
        sid = lax.axis_index("s")
        mybase = (sid * NC + cid) * TPT
        pltpu.sync_copy(z_hbm, cnt)
        ones = jnp.ones((16,), jnp.float32)

        def chunk_body(ch, _):
            pltpu.sync_copy(dst_hbm.at[pl.ds(ch * SE, SE)], dst_ch)

            def scan_body(i, _):
                ld = dst_ch[pl.ds(i * 16, 16)] - mybase
                m = (ld >= 0) & (ld < TPT)
                plsc.addupdate_scatter(cnt, [ld], ones, mask=m)
                return 0

            lax.fori_loop(0, SE // 16, scan_body, 0)
            return 0

        lax.fori_loop(0, NCH, chunk_body, 0)
        pltpu.sync_copy(cnt.at[pl.ds(0, TPT)], out_hbm.at[pl.ds(mybase, TPT)])

    return deg_kernel(dst, zeros_init)


def _aggregate(h, ei):
    assert h.shape == (N_NODES, AGG_C)
    zeros_init = jnp.zeros((TPT + 8, AGG_C), jnp.float32)
    out = _sc_aggregate(h, ei[0], ei[1], zeros_init)
    return out[:N_NODES]


def _degree(ei):
    zeros_init = jnp.zeros((TPT + 16,), jnp.float32)
    return _sc_degree(ei[0], ei[1], zeros_init)[:N_NODES].reshape(N_NODES, 1)


def _bn_coeffs(s_ref, q_ref, g_ref, b_ref):
    mu = s_ref[0:1, :] * (1.0 / N_NODES)
    var = q_ref[0:1, :] * (1.0 / N_NODES) - mu * mu
    scale = lax.rsqrt(var + EPS) * g_ref[...]
    shift = b_ref[...] - mu * scale
    return scale, shift


def _accum_stats(i, r, s_ref, q_ref):
    @pl.when(i == 0)
    def _():
        s_ref[...] = jnp.zeros_like(s_ref)
        q_ref[...] = jnp.zeros_like(q_ref)

    s_ref[...] += jnp.broadcast_to(jnp.sum(r, 0, keepdims=True), s_ref.shape)
    q_ref[...] += jnp.broadcast_to(jnp.sum(r * r, 0, keepdims=True), q_ref.shape)


def _dot(a, b):
    return jnp.dot(a, b, preferred_element_type=jnp.float32,
                   precision=lax.Precision.HIGHEST)


def _row_spec(c):
    return pl.BlockSpec((R_BLK, c), lambda i: (i, 0))


def _full_spec(shape):
    return pl.BlockSpec(shape, lambda i: tuple(0 for _ in shape))


def _stats_spec(c):
    return pl.BlockSpec((8, c), lambda i: (0, 0))


def _tc_colstats(x):
    c = x.shape[1]

    def body(x_ref, s_ref, q_ref):
        _accum_stats(pl.program_id(0), x_ref[...], s_ref, q_ref)

    return pl.pallas_call(
        body,
        grid=(N_BLKS,),
        in_specs=[_row_spec(c)],
        out_specs=[_stats_spec(c), _stats_spec(c)],
        out_shape=[jax.ShapeDtypeStruct((8, c), jnp.float32)] * 2,
    )(x)


def _tc_make_y(x, s0, q0, g0, b0, deg):
    c = x.shape[1]

    def body(x_ref, s_ref, q_ref, g_ref, b_ref, d_ref, y_ref):
        scale, shift = _bn_coeffs(s_ref, q_ref, g_ref, b_ref)
        dinv = lax.rsqrt(d_ref[:, 0:1] + 1.0)
        y_ref[...] = dinv * (x_ref[...] * scale + shift)

    return pl.pallas_call(
        body,
        grid=(N_BLKS,),
        in_specs=[_row_spec(c), _stats_spec(c), _stats_spec(c),
                  _full_spec((1, c)), _full_spec((1, c)),
                  pl.BlockSpec((R_BLK, 1), lambda i: (i, 0))],
        out_specs=[_row_spec(c)],
        out_shape=[jax.ShapeDtypeStruct((N_NODES, c), jnp.float32)],
    )(x, s0, q0, g0, b0, deg)[0]


def _tc_layer1(agg1, y, deg, w1, b1):
    cin, cout = w1.shape

    def body(a_ref, y_ref, d_ref, w_ref, b_ref, r_ref, s_ref, q_ref):
        dinv = lax.rsqrt(d_ref[:, 0:1] + 1.0)
        u = dinv * (a_ref[...] + y_ref[...])
        r = jnp.maximum(_dot(u, w_ref[...]) + b_ref[...], 0.0)
        r_ref[...] = r
        _accum_stats(pl.program_id(0), r, s_ref, q_ref)

    return pl.pallas_call(
        body,
        grid=(N_BLKS,),
        in_specs=[_row_spec(cin), _row_spec(cin),
                  pl.BlockSpec((R_BLK, 1), lambda i: (i, 0)),
                  _full_spec((cin, cout)), _full_spec((1, cout))],
        out_specs=[_row_spec(cout), _stats_spec(cout), _stats_spec(cout)],
        out_shape=[jax.ShapeDtypeStruct((N_NODES, cout), jnp.float32),
                   jax.ShapeDtypeStruct((8, cout), jnp.float32),
                   jax.ShapeDtypeStruct((8, cout), jnp.float32)],
    )(agg1, y, deg, w1, b1)


def _tc_bn_apply(r, s, q, g, b):
    c = r.shape[1]

    def body(r_ref, s_ref, q_ref, g_ref, b_ref, o_ref):
        scale, shift = _bn_coeffs(s_ref, q_ref, g_ref, b_ref)
        o_ref[...] = r_ref[...] * scale + shift

    return pl.pallas_call(
        body,
        grid=(N_BLKS,),
        in_specs=[_row_spec(c), _stats_spec(c), _stats_spec(c),
                  _full_spec((1, c)), _full_spec((1, c))],
        out_specs=[_row_spec(c)],
        out_shape=[jax.ShapeDtypeStruct((N_NODES, c), jnp.float32)],
    )(r, s, q, g, b)[0]


def _tc_graphconv(agg, h, w_rel, w_root, b):
    cin, cout = w_rel.shape

    def body(a_ref, h_ref, wr_ref, wo_ref, b_ref, r_ref, s_ref, q_ref):
        t = _dot(a_ref[...], wr_ref[...]) + _dot(h_ref[...], wo_ref[...])
        r = jnp.maximum(t + b_ref[...], 0.0)
        r_ref[...] = r
        _accum_stats(pl.program_id(0), r, s_ref, q_ref)

    return pl.pallas_call(
        body,
        grid=(N_BLKS,),
        in_specs=[_row_spec(cin), _row_spec(cin),
                  _full_spec((cin, cout)), _full_spec((cin, cout)),
                  _full_spec((1, cout))],
        out_specs=[_row_spec(cout), _stats_spec(cout), _stats_spec(cout)],
        out_shape=[jax.ShapeDtypeStruct((N_NODES, cout), jnp.float32),
                   jax.ShapeDtypeStruct((8, cout), jnp.float32),
                   jax.ShapeDtypeStruct((8, cout), jnp.float32)],
    )(agg, h, w_rel, w_root, b)


def _tc_bn_matmul(r, s, q, g, b, w):
    c = r.shape[1]
    cout = w.shape[1]

    def body(r_ref, s_ref, q_ref, g_ref, b_ref, w_ref, h_ref, p_ref):
        scale, shift = _bn_coeffs(s_ref, q_ref, g_ref, b_ref)
        h = r_ref[...] * scale + shift
        h_ref[...] = h
        p_ref[...] = _dot(h, w_ref[...])

    return pl.pallas_call(
        body,
        grid=(N_BLKS,),
        in_specs=[_row_spec(c), _stats_spec(c), _stats_spec(c),
                  _full_spec((1, c)), _full_spec((1, c)),
                  _full_spec((c, cout))],
        out_specs=[_row_spec(c), _row_spec(cout)],
        out_shape=[jax.ShapeDtypeStruct((N_NODES, c), jnp.float32),
                   jax.ShapeDtypeStruct((N_NODES, cout), jnp.float32)],
    )(r, s, q, g, b, w)


def _tc_layer3_tail(agg3, h2, w_root, b3):
    cin, cout = w_root.shape

    def body(a_ref, h_ref, w_ref, b_ref, r_ref, s_ref, q_ref):
        t = a_ref[...] + _dot(h_ref[...], w_ref[...])
        r = jnp.maximum(t + b_ref[...], 0.0)
        r_ref[...] = r
        _accum_stats(pl.program_id(0), r, s_ref, q_ref)

    return pl.pallas_call(
        body,
        grid=(N_BLKS,),
        in_specs=[_row_spec(cout), _row_spec(cin),
                  _full_spec((cin, cout)), _full_spec((1, cout))],
        out_specs=[_row_spec(cout), _stats_spec(cout), _stats_spec(cout)],
        out_shape=[jax.ShapeDtypeStruct((N_NODES, cout), jnp.float32),
                   jax.ShapeDtypeStruct((8, cout), jnp.float32),
                   jax.ShapeDtypeStruct((8, cout), jnp.float32)],
    )(agg3, h2, w_root, b3)


def kernel(x, edge_index, gamma0, beta0, W1, b1, gamma1, beta1,
           W2_rel, b2, W2_root, gamma2, beta2,
           W3_rel, b3, W3_root, gamma3, beta3):
    ei = edge_index.astype(jnp.int32)
    g0, b0 = gamma0.reshape(1, -1), beta0.reshape(1, -1)
    g1, b1r = gamma1.reshape(1, -1), b1.reshape(1, -1)
    be1 = beta1.reshape(1, -1)
    g2, b2r, be2 = gamma2.reshape(1, -1), b2.reshape(1, -1), beta2.reshape(1, -1)
    g3, b3r, be3 = gamma3.reshape(1, -1), b3.reshape(1, -1), beta3.reshape(1, -1)

    deg = _degree(ei)

    s0, q0 = _tc_colstats(x)
    y = _tc_make_y(x, s0, q0, g0, b0, deg)
    agg1 = _aggregate(y, ei)
    r1, s1, q1 = _tc_layer1(agg1, y, deg, W1, b1r)

    h1 = _tc_bn_apply(r1, s1, q1, g1, be1)
    agg2 = jnp.concatenate(
        [_aggregate(h1[:, :AGG_C], ei), _aggregate(h1[:, AGG_C:], ei)], axis=1)
    r2, s2, q2 = _tc_graphconv(agg2, h1, W2_rel, W2_root, b2r)

    h2, p = _tc_bn_matmul(r2, s2, q2, g2, be2, W3_rel)
    agg3 = _aggregate(p, ei)
    r3, s3, q3 = _tc_layer3_tail(agg3, h2, W3_root, b3r)

    return _tc_bn_apply(r3, s3, q3, g3, be3)


# speedup vs baseline: 4.4329x; 1.4037x over previous
import functools

import jax
import jax.numpy as jnp
from jax import lax
from jax.experimental import pallas as pl
from jax.experimental.pallas import tpu as pltpu
from jax.experimental.pallas import tpu_sc as plsc

N_NODES = 10000
N_EDGES = 160000
NC, NS = 2, 16
R_BLK = 2000
N_BLKS = N_NODES // R_BLK
EPS = 1e-5


AGG_C = 256
NW = NC * NS
TPT = 320
N_PAD = NW * TPT
SE = 2000
NCH = N_EDGES // SE
GE = 64
FL = 2048
EMAX = N_EDGES + FL
PACK_BITS = 14

_SC_MESH = dict(core_axis_name="c", subcore_axis_name="s")
_SC_CP = pltpu.CompilerParams(needs_layout_passes=False)


@jax.jit
def _sc_bucket(src, dst):
    @functools.partial(
        pl.kernel,
        out_type=(jax.ShapeDtypeStruct((NW * EMAX,), jnp.int32),
                  jax.ShapeDtypeStruct((NW * 16,), jnp.int32),
                  jax.ShapeDtypeStruct((N_PAD,), jnp.float32)),
        mesh=plsc.VectorSubcoreMesh(**_SC_MESH),
        compiler_params=_SC_CP,
        scratch_types=[
            pltpu.VMEM((SE,), jnp.int32),
            pltpu.VMEM((SE,), jnp.int32),
            pltpu.VMEM((FL + SE + 80,), jnp.int32),
            pltpu.VMEM((16,), jnp.int32),
            pltpu.VMEM((TPT + 16,), jnp.float32),
        ],
    )
    def bucket_kernel(src_hbm, dst_hbm, list_hbm, cnt_hbm, deg_hbm,
                      src_ch, dst_ch, sbuf, cstage, cnt):
        cid = lax.axis_index("c")
        sid = lax.axis_index("s")
        w = sid * NC + cid
        mybase = pl.multiple_of(w * TPT, 8)
        lbase = pl.multiple_of(w * EMAX, 8)
        lane = lax.iota(jnp.int32, 16)
        lane_masks = [lane == k for k in range(16)]
        ones = jnp.ones((16,), jnp.float32)
        dump_pack = jnp.full((16,), TPT << PACK_BITS, jnp.int32)

        @pl.loop(0, (TPT + 16) // 16)
        def _(i):
            cnt[pl.ds(i * 16, 16)] = jnp.zeros((16,), jnp.float32)

        def deg_region(lo, hi):
            def dbody(i, _):
                ld = sbuf[pl.ds(lo + i * 16, 16)] >> PACK_BITS
                for k in range(16):
                    plsc.addupdate_scatter(cnt, [ld], ones,
                                           mask=(lane_masks[k]) & (ld < TPT))
                return 0

            lax.fori_loop(0, (hi - lo) // 16, dbody, 0)

        def chunk_body(ch, carry):
            ptr, gout = carry
            e0 = ch * SE
            pltpu.sync_copy(src_hbm.at[pl.ds(e0, SE)], src_ch)
            pltpu.sync_copy(dst_hbm.at[pl.ds(e0, SE)], dst_ch)

            def scan_body(i, ptr):
                vd = dst_ch[pl.ds(i * 16, 16)]
                vs = src_ch[pl.ds(i * 16, 16)]
                ld = vd - mybase
                m = (ld >= 0) & (ld < TPT)
                packed = vs | (ld << PACK_BITS)
                plsc.store_compressed(sbuf.at[pl.ds(ptr, 16)], packed, mask=m)
                return ptr + jnp.max(plsc.all_reduce_population_count(m))

            ptr = lax.fori_loop(0, SE // 16, scan_body, ptr)

            @pl.when(ptr >= FL)
            def _():
                deg_region(0, FL)
                pltpu.sync_copy(sbuf.at[pl.ds(0, FL)],
                                list_hbm.at[pl.ds(pl.multiple_of(lbase + gout, 8), FL)])
                tail = ptr - FL

                def mv(i, _):
                    sbuf[pl.ds(i * 16, 16)] = sbuf[pl.ds(FL + i * 16, 16)]
                    return 0

                lax.fori_loop(0, (tail + 15) // 16, mv, 0)

            flushed = jnp.where(ptr >= FL, FL, 0)
            return ptr - flushed, gout + flushed

        ptr, gout = lax.fori_loop(0, NCH, chunk_body,
                                  (jnp.int32(0), jnp.int32(0)))
        padded = ((ptr + 2 * GE - 1) // (2 * GE)) * (2 * GE)

        def padfill(i, _):
            sbuf[pl.ds(ptr + i * 16, 16)] = dump_pack
            return 0

        lax.fori_loop(0, (padded - ptr + 15) // 16, padfill, 0)

        @pl.when(padded > 0)
        def _():
            deg_region(0, padded)
            pltpu.sync_copy(sbuf.at[pl.ds(0, FL)],
                            list_hbm.at[pl.ds(pl.multiple_of(lbase + gout, 8), FL)])

        cstage[pl.ds(0, 16)] = jnp.broadcast_to(gout + padded, (16,))
        pltpu.sync_copy(cstage, cnt_hbm.at[pl.ds(pl.multiple_of(w * 16, 8), 16)])
        pltpu.sync_copy(cnt.at[pl.ds(0, TPT)], deg_hbm.at[pl.ds(mybase, TPT)])

    return bucket_kernel(src, dst)


@jax.jit
def _sc_aggregate(h, elist, counts, zeros_init):
    c = AGG_C

    @functools.partial(
        pl.kernel,
        out_type=jax.ShapeDtypeStruct((N_PAD, c), jnp.float32),
        mesh=plsc.VectorSubcoreMesh(**_SC_MESH),
        compiler_params=_SC_CP,
        scratch_types=[
            pltpu.VMEM((16,), jnp.int32),
            pltpu.VMEM((GE,), jnp.int32),
            pltpu.VMEM((GE,), jnp.int32),
            pltpu.VMEM((GE,), jnp.int32),
            pltpu.VMEM((GE,), jnp.int32),
            pltpu.VMEM((GE, c), jnp.float32),
            pltpu.VMEM((GE, c), jnp.float32),
            pltpu.VMEM((TPT + 8, c), jnp.float32),
            pltpu.SemaphoreType.DMA,
            pltpu.SemaphoreType.DMA,
        ],
    )
    def agg_kernel(h_hbm, list_hbm, cnt_hbm, z_hbm, out_hbm,
                   cbuf, lg_a, lg_b, gi_a, gi_b, rb_a, rb_b, acc,
                   sem_a, sem_b):
        cid = lax.axis_index("c")
        sid = lax.axis_index("s")
        w = sid * NC + cid
        mybase = pl.multiple_of(w * TPT, 8)
        lbase = pl.multiple_of(w * EMAX, 8)
        lane = lax.iota(jnp.int32, 16)
        pltpu.sync_copy(z_hbm, acc)
        pltpu.sync_copy(cnt_hbm.at[pl.ds(pl.multiple_of(w * 16, 8), 16)], cbuf)
        ng = jnp.max(cbuf[pl.ds(0, 16)]) // GE

        def issue(g, lg, gi, rb, sem):
            pltpu.sync_copy(list_hbm.at[pl.ds(pl.multiple_of(lbase + g * GE, 8), GE)], lg)
            for k in range(GE // 16):
                gi[pl.ds(k * 16, 16)] = lg[pl.ds(k * 16, 16)] & ((1 << PACK_BITS) - 1)
            pltpu.async_copy(h_hbm.at[gi], rb, sem)

        def wait(rb, sem):
            pltpu.make_async_copy(h_hbm.at[pl.ds(0, GE)], rb, sem).wait()

        def process(lg, rb):
            @pl.loop(0, GE // 16)
            def _(q):
                vlad = lg[pl.ds(q * 16, 16)] >> PACK_BITS
                for r2 in range(16):
                    rowv = vlad.at[jnp.full((16,), r2, jnp.int32)].get(
                        mode="promise_in_bounds")
                    for j in range(c // 16):
                        plsc.addupdate_scatter(
                            acc, [rowv, j * 16 + lane],
                            rb[q * 16 + r2, pl.ds(j * 16, 16)])

        @pl.when(ng > 0)
        def _():
            issue(0, lg_a, gi_a, rb_a, sem_a)

            def pair(i, _):
                g0 = 2 * i
                issue(g0 + 1, lg_b, gi_b, rb_b, sem_b)
                wait(rb_a, sem_a)
                process(lg_a, rb_a)

                @pl.when(g0 + 2 < ng)
                def _():
                    issue(g0 + 2, lg_a, gi_a, rb_a, sem_a)

                wait(rb_b, sem_b)
                process(lg_b, rb_b)
                return 0

            lax.fori_loop(0, ng // 2, pair, 0)

        pltpu.sync_copy(acc.at[pl.ds(0, TPT)], out_hbm.at[pl.ds(mybase, TPT)])

    return agg_kernel(h, elist, counts, zeros_init)


def _bucket(ei):
    return _sc_bucket(ei[0], ei[1])


def _aggregate(h, elist, counts):
    assert h.shape == (N_NODES, AGG_C)
    zeros_init = jnp.zeros((TPT + 8, AGG_C), jnp.float32)
    out = _sc_aggregate(h, elist, counts, zeros_init)
    return out[:N_NODES]


def _bn_coeffs(s_ref, q_ref, g_ref, b_ref):
    mu = s_ref[0:1, :] * (1.0 / N_NODES)
    var = q_ref[0:1, :] * (1.0 / N_NODES) - mu * mu
    scale = lax.rsqrt(var + EPS) * g_ref[...]
    shift = b_ref[...] - mu * scale
    return scale, shift


def _accum_stats(i, r, s_ref, q_ref):
    @pl.when(i == 0)
    def _():
        s_ref[...] = jnp.zeros_like(s_ref)
        q_ref[...] = jnp.zeros_like(q_ref)

    s_ref[...] += jnp.broadcast_to(jnp.sum(r, 0, keepdims=True), s_ref.shape)
    q_ref[...] += jnp.broadcast_to(jnp.sum(r * r, 0, keepdims=True), q_ref.shape)


def _dot(a, b):
    return jnp.dot(a, b, preferred_element_type=jnp.float32,
                   precision=lax.Precision.HIGHEST)


def _row_spec(c):
    return pl.BlockSpec((R_BLK, c), lambda i: (i, 0))


def _full_spec(shape):
    return pl.BlockSpec(shape, lambda i: tuple(0 for _ in shape))


def _stats_spec(c):
    return pl.BlockSpec((8, c), lambda i: (0, 0))


def _tc_colstats(x):
    c = x.shape[1]

    def body(x_ref, s_ref, q_ref):
        _accum_stats(pl.program_id(0), x_ref[...], s_ref, q_ref)

    return pl.pallas_call(
        body,
        grid=(N_BLKS,),
        in_specs=[_row_spec(c)],
        out_specs=[_stats_spec(c), _stats_spec(c)],
        out_shape=[jax.ShapeDtypeStruct((8, c), jnp.float32)] * 2,
    )(x)


def _tc_make_y(x, s0, q0, g0, b0, deg):
    c = x.shape[1]

    def body(x_ref, s_ref, q_ref, g_ref, b_ref, d_ref, y_ref):
        scale, shift = _bn_coeffs(s_ref, q_ref, g_ref, b_ref)
        dinv = lax.rsqrt(d_ref[:, 0:1] + 1.0)
        y_ref[...] = dinv * (x_ref[...] * scale + shift)

    return pl.pallas_call(
        body,
        grid=(N_BLKS,),
        in_specs=[_row_spec(c), _stats_spec(c), _stats_spec(c),
                  _full_spec((1, c)), _full_spec((1, c)),
                  pl.BlockSpec((R_BLK, 1), lambda i: (i, 0))],
        out_specs=[_row_spec(c)],
        out_shape=[jax.ShapeDtypeStruct((N_NODES, c), jnp.float32)],
    )(x, s0, q0, g0, b0, deg)[0]


def _tc_layer1(agg1, y, deg, w1, b1):
    cin, cout = w1.shape

    def body(a_ref, y_ref, d_ref, w_ref, b_ref, r_ref, s_ref, q_ref):
        dinv = lax.rsqrt(d_ref[:, 0:1] + 1.0)
        u = dinv * (a_ref[...] + y_ref[...])
        r = jnp.maximum(_dot(u, w_ref[...]) + b_ref[...], 0.0)
        r_ref[...] = r
        _accum_stats(pl.program_id(0), r, s_ref, q_ref)

    return pl.pallas_call(
        body,
        grid=(N_BLKS,),
        in_specs=[_row_spec(cin), _row_spec(cin),
                  pl.BlockSpec((R_BLK, 1), lambda i: (i, 0)),
                  _full_spec((cin, cout)), _full_spec((1, cout))],
        out_specs=[_row_spec(cout), _stats_spec(cout), _stats_spec(cout)],
        out_shape=[jax.ShapeDtypeStruct((N_NODES, cout), jnp.float32),
                   jax.ShapeDtypeStruct((8, cout), jnp.float32),
                   jax.ShapeDtypeStruct((8, cout), jnp.float32)],
    )(agg1, y, deg, w1, b1)


def _tc_bn_apply(r, s, q, g, b):
    c = r.shape[1]

    def body(r_ref, s_ref, q_ref, g_ref, b_ref, o_ref):
        scale, shift = _bn_coeffs(s_ref, q_ref, g_ref, b_ref)
        o_ref[...] = r_ref[...] * scale + shift

    return pl.pallas_call(
        body,
        grid=(N_BLKS,),
        in_specs=[_row_spec(c), _stats_spec(c), _stats_spec(c),
                  _full_spec((1, c)), _full_spec((1, c))],
        out_specs=[_row_spec(c)],
        out_shape=[jax.ShapeDtypeStruct((N_NODES, c), jnp.float32)],
    )(r, s, q, g, b)[0]


def _tc_graphconv(agg, h, w_rel, w_root, b):
    cin, cout = w_rel.shape

    def body(a_ref, h_ref, wr_ref, wo_ref, b_ref, r_ref, s_ref, q_ref):
        t = _dot(a_ref[...], wr_ref[...]) + _dot(h_ref[...], wo_ref[...])
        r = jnp.maximum(t + b_ref[...], 0.0)
        r_ref[...] = r
        _accum_stats(pl.program_id(0), r, s_ref, q_ref)

    return pl.pallas_call(
        body,
        grid=(N_BLKS,),
        in_specs=[_row_spec(cin), _row_spec(cin),
                  _full_spec((cin, cout)), _full_spec((cin, cout)),
                  _full_spec((1, cout))],
        out_specs=[_row_spec(cout), _stats_spec(cout), _stats_spec(cout)],
        out_shape=[jax.ShapeDtypeStruct((N_NODES, cout), jnp.float32),
                   jax.ShapeDtypeStruct((8, cout), jnp.float32),
                   jax.ShapeDtypeStruct((8, cout), jnp.float32)],
    )(agg, h, w_rel, w_root, b)


def _tc_bn_matmul(r, s, q, g, b, w):
    c = r.shape[1]
    cout = w.shape[1]

    def body(r_ref, s_ref, q_ref, g_ref, b_ref, w_ref, h_ref, p_ref):
        scale, shift = _bn_coeffs(s_ref, q_ref, g_ref, b_ref)
        h = r_ref[...] * scale + shift
        h_ref[...] = h
        p_ref[...] = _dot(h, w_ref[...])

    return pl.pallas_call(
        body,
        grid=(N_BLKS,),
        in_specs=[_row_spec(c), _stats_spec(c), _stats_spec(c),
                  _full_spec((1, c)), _full_spec((1, c)),
                  _full_spec((c, cout))],
        out_specs=[_row_spec(c), _row_spec(cout)],
        out_shape=[jax.ShapeDtypeStruct((N_NODES, c), jnp.float32),
                   jax.ShapeDtypeStruct((N_NODES, cout), jnp.float32)],
    )(r, s, q, g, b, w)


def _tc_layer3_tail(agg3, h2, w_root, b3):
    cin, cout = w_root.shape

    def body(a_ref, h_ref, w_ref, b_ref, r_ref, s_ref, q_ref):
        t = a_ref[...] + _dot(h_ref[...], w_ref[...])
        r = jnp.maximum(t + b_ref[...], 0.0)
        r_ref[...] = r
        _accum_stats(pl.program_id(0), r, s_ref, q_ref)

    return pl.pallas_call(
        body,
        grid=(N_BLKS,),
        in_specs=[_row_spec(cout), _row_spec(cin),
                  _full_spec((cin, cout)), _full_spec((1, cout))],
        out_specs=[_row_spec(cout), _stats_spec(cout), _stats_spec(cout)],
        out_shape=[jax.ShapeDtypeStruct((N_NODES, cout), jnp.float32),
                   jax.ShapeDtypeStruct((8, cout), jnp.float32),
                   jax.ShapeDtypeStruct((8, cout), jnp.float32)],
    )(agg3, h2, w_root, b3)


def kernel(x, edge_index, gamma0, beta0, W1, b1, gamma1, beta1,
           W2_rel, b2, W2_root, gamma2, beta2,
           W3_rel, b3, W3_root, gamma3, beta3):
    ei = edge_index.astype(jnp.int32)
    g0, b0 = gamma0.reshape(1, -1), beta0.reshape(1, -1)
    g1, b1r = gamma1.reshape(1, -1), b1.reshape(1, -1)
    be1 = beta1.reshape(1, -1)
    g2, b2r, be2 = gamma2.reshape(1, -1), b2.reshape(1, -1), beta2.reshape(1, -1)
    g3, b3r, be3 = gamma3.reshape(1, -1), b3.reshape(1, -1), beta3.reshape(1, -1)

    elist, counts, degflat = _bucket(ei)
    deg = degflat[:N_NODES].reshape(N_NODES, 1)

    s0, q0 = _tc_colstats(x)
    y = _tc_make_y(x, s0, q0, g0, b0, deg)
    agg1 = _aggregate(y, elist, counts)
    r1, s1, q1 = _tc_layer1(agg1, y, deg, W1, b1r)

    h1 = _tc_bn_apply(r1, s1, q1, g1, be1)
    agg2 = jnp.concatenate(
        [_aggregate(h1[:, :AGG_C], elist, counts),
         _aggregate(h1[:, AGG_C:], elist, counts)], axis=1)
    r2, s2, q2 = _tc_graphconv(agg2, h1, W2_rel, W2_root, b2r)

    h2, p = _tc_bn_matmul(r2, s2, q2, g2, be2, W3_rel)
    agg3 = _aggregate(p, elist, counts)
    r3, s3, q3 = _tc_layer3_tail(agg3, h2, W3_root, b3r)

    return _tc_bn_apply(r3, s3, q3, g3, be3)


# speedup vs baseline: 4.6579x; 1.0508x over previous
import functools

import jax
import jax.numpy as jnp
from jax import lax
from jax.experimental import pallas as pl
from jax.experimental.pallas import tpu as pltpu
from jax.experimental.pallas import tpu_sc as plsc

N_NODES = 10000
N_EDGES = 160000
NC, NS = 2, 16
R_BLK = 2000
N_BLKS = N_NODES // R_BLK
EPS = 1e-5


AGG_C = 256
NW = NC * NS
TPT = 320
N_PAD = NW * TPT
SE = 2000
NCH = N_EDGES // SE
GE = 64
FL = 2048
EMAX = N_EDGES + FL
PACK_BITS = 14

_SC_MESH = dict(core_axis_name="c", subcore_axis_name="s")
_SC_CP = pltpu.CompilerParams(needs_layout_passes=False)


@jax.jit
def _sc_bucket(src, dst):
    @functools.partial(
        pl.kernel,
        out_type=(jax.ShapeDtypeStruct((NW * EMAX,), jnp.int32),
                  jax.ShapeDtypeStruct((NW * 16,), jnp.int32),
                  jax.ShapeDtypeStruct((N_PAD,), jnp.float32)),
        mesh=plsc.VectorSubcoreMesh(**_SC_MESH),
        compiler_params=_SC_CP,
        scratch_types=[
            pltpu.VMEM((SE,), jnp.int32),
            pltpu.VMEM((SE,), jnp.int32),
            pltpu.VMEM((FL + SE + 80,), jnp.int32),
            pltpu.VMEM((16,), jnp.int32),
            pltpu.VMEM((TPT + 16,), jnp.float32),
        ],
    )
    def bucket_kernel(src_hbm, dst_hbm, list_hbm, cnt_hbm, deg_hbm,
                      src_ch, dst_ch, sbuf, cstage, cnt):
        cid = lax.axis_index("c")
        sid = lax.axis_index("s")
        w = sid * NC + cid
        mybase = pl.multiple_of(w * TPT, 8)
        lbase = pl.multiple_of(w * EMAX, 8)
        lane = lax.iota(jnp.int32, 16)
        lane_masks = [lane == k for k in range(16)]
        ones = jnp.ones((16,), jnp.float32)
        dump_pack = jnp.full((16,), TPT << PACK_BITS, jnp.int32)

        @pl.loop(0, (TPT + 16) // 16)
        def _(i):
            cnt[pl.ds(i * 16, 16)] = jnp.zeros((16,), jnp.float32)

        def deg_region(lo, hi):
            def dbody(i, _):
                ld = sbuf[pl.ds(lo + i * 16, 16)] >> PACK_BITS
                for k in range(16):
                    plsc.addupdate_scatter(cnt, [ld], ones,
                                           mask=(lane_masks[k]) & (ld < TPT))
                return 0

            lax.fori_loop(0, (hi - lo) // 16, dbody, 0)

        def chunk_body(ch, carry):
            ptr, gout = carry
            e0 = ch * SE
            pltpu.sync_copy(src_hbm.at[pl.ds(e0, SE)], src_ch)
            pltpu.sync_copy(dst_hbm.at[pl.ds(e0, SE)], dst_ch)

            def scan_body(i, ptr):
                vd = dst_ch[pl.ds(i * 16, 16)]
                vs = src_ch[pl.ds(i * 16, 16)]
                ld = vd - mybase
                m = (ld >= 0) & (ld < TPT)
                packed = vs | (ld << PACK_BITS)
                plsc.store_compressed(sbuf.at[pl.ds(ptr, 16)], packed, mask=m)
                return ptr + jnp.max(plsc.all_reduce_population_count(m))

            ptr = lax.fori_loop(0, SE // 16, scan_body, ptr)

            @pl.when(ptr >= FL)
            def _():
                deg_region(0, FL)
                pltpu.sync_copy(sbuf.at[pl.ds(0, FL)],
                                list_hbm.at[pl.ds(pl.multiple_of(lbase + gout, 8), FL)])
                tail = ptr - FL

                def mv(i, _):
                    sbuf[pl.ds(i * 16, 16)] = sbuf[pl.ds(FL + i * 16, 16)]
                    return 0

                lax.fori_loop(0, (tail + 15) // 16, mv, 0)

            flushed = jnp.where(ptr >= FL, FL, 0)
            return ptr - flushed, gout + flushed

        ptr, gout = lax.fori_loop(0, NCH, chunk_body,
                                  (jnp.int32(0), jnp.int32(0)))
        padded = ((ptr + 2 * GE - 1) // (2 * GE)) * (2 * GE)

        def padfill(i, _):
            sbuf[pl.ds(ptr + i * 16, 16)] = dump_pack
            return 0

        lax.fori_loop(0, (padded - ptr + 15) // 16, padfill, 0)

        @pl.when(padded > 0)
        def _():
            deg_region(0, padded)
            pltpu.sync_copy(sbuf.at[pl.ds(0, FL)],
                            list_hbm.at[pl.ds(pl.multiple_of(lbase + gout, 8), FL)])

        cstage[pl.ds(0, 16)] = jnp.broadcast_to(gout + padded, (16,))
        pltpu.sync_copy(cstage, cnt_hbm.at[pl.ds(pl.multiple_of(w * 16, 8), 16)])
        pltpu.sync_copy(cnt.at[pl.ds(0, TPT)], deg_hbm.at[pl.ds(mybase, TPT)])

    return bucket_kernel(src, dst)


@jax.jit
def _sc_aggregate(h, elist, counts, zeros_init):
    c = AGG_C
    CH = 2048
    CHG = CH // GE

    @functools.partial(
        pl.kernel,
        out_type=jax.ShapeDtypeStruct((N_PAD, c), jnp.float32),
        mesh=plsc.VectorSubcoreMesh(**_SC_MESH),
        compiler_params=_SC_CP,
        scratch_types=[
            pltpu.VMEM((16,), jnp.int32),
            pltpu.VMEM((CH,), jnp.int32),
            pltpu.VMEM((GE,), jnp.int32),
            pltpu.VMEM((GE,), jnp.int32),
            pltpu.VMEM((GE, c), jnp.float32),
            pltpu.VMEM((GE, c), jnp.float32),
            pltpu.VMEM((TPT + 8, c), jnp.float32),
            pltpu.SemaphoreType.DMA,
            pltpu.SemaphoreType.DMA,
        ],
    )
    def agg_kernel(h_hbm, list_hbm, cnt_hbm, z_hbm, out_hbm,
                   cbuf, lbuf, gi_a, gi_b, rb_a, rb_b, acc, sem_a, sem_b):
        cid = lax.axis_index("c")
        sid = lax.axis_index("s")
        w = sid * NC + cid
        mybase = pl.multiple_of(w * TPT, 8)
        lbase = pl.multiple_of(w * EMAX, 8)
        lane = lax.iota(jnp.int32, 16)
        pltpu.sync_copy(z_hbm, acc)
        pltpu.sync_copy(cnt_hbm.at[pl.ds(pl.multiple_of(w * 16, 8), 16)], cbuf)
        ng = jnp.max(cbuf[pl.ds(0, 16)]) // GE

        def issue(gl, gi, rb, sem):
            for k in range(GE // 16):
                gi[pl.ds(k * 16, 16)] = (lbuf[pl.ds(gl * GE + k * 16, 16)]
                                         & ((1 << PACK_BITS) - 1))
            pltpu.async_copy(h_hbm.at[gi], rb, sem)

        def wait(rb, sem):
            pltpu.make_async_copy(h_hbm.at[pl.ds(0, GE)], rb, sem).wait()

        def process(gl, rb):
            @pl.loop(0, GE // 16)
            def _(q):
                vlad = lbuf[pl.ds(gl * GE + q * 16, 16)] >> PACK_BITS
                for r2 in range(16):
                    rowv = vlad.at[jnp.full((16,), r2, jnp.int32)].get(
                        mode="promise_in_bounds")
                    for j in range(c // 16):
                        plsc.addupdate_scatter(
                            acc, [rowv, j * 16 + lane],
                            rb[q * 16 + r2, pl.ds(j * 16, 16)])

        def chunk(lc, _):
            ngc = jnp.minimum(CHG, ng - lc * CHG)
            pltpu.sync_copy(
                list_hbm.at[pl.ds(pl.multiple_of(lbase + lc * CH, 8), CH)],
                lbuf)
            issue(0, gi_a, rb_a, sem_a)

            def pair(i, _):
                g0 = 2 * i
                issue(g0 + 1, gi_b, rb_b, sem_b)
                wait(rb_a, sem_a)
                process(g0, rb_a)

                @pl.when(g0 + 2 < ngc)
                def _():
                    issue(g0 + 2, gi_a, rb_a, sem_a)

                wait(rb_b, sem_b)
                process(g0 + 1, rb_b)
                return 0

            lax.fori_loop(0, ngc // 2, pair, 0)
            return 0

        @pl.when(ng > 0)
        def _():
            lax.fori_loop(0, (ng + CHG - 1) // CHG, chunk, 0)

        pltpu.sync_copy(acc.at[pl.ds(0, TPT)], out_hbm.at[pl.ds(mybase, TPT)])

    return agg_kernel(h, elist, counts, zeros_init)


def _bucket(ei):
    return _sc_bucket(ei[0], ei[1])


def _aggregate(h, elist, counts):
    assert h.shape == (N_NODES, AGG_C)
    zeros_init = jnp.zeros((TPT + 8, AGG_C), jnp.float32)
    out = _sc_aggregate(h, elist, counts, zeros_init)
    return out[:N_NODES]


def _bn_coeffs(s_ref, q_ref, g_ref, b_ref):
    mu = s_ref[0:1, :] * (1.0 / N_NODES)
    var = q_ref[0:1, :] * (1.0 / N_NODES) - mu * mu
    scale = lax.rsqrt(var + EPS) * g_ref[...]
    shift = b_ref[...] - mu * scale
    return scale, shift


def _accum_stats(i, r, s_ref, q_ref):
    @pl.when(i == 0)
    def _():
        s_ref[...] = jnp.zeros_like(s_ref)
        q_ref[...] = jnp.zeros_like(q_ref)

    s_ref[...] += jnp.broadcast_to(jnp.sum(r, 0, keepdims=True), s_ref.shape)
    q_ref[...] += jnp.broadcast_to(jnp.sum(r * r, 0, keepdims=True), q_ref.shape)


def _dot(a, b):
    return jnp.dot(a, b, preferred_element_type=jnp.float32,
                   precision=lax.Precision.HIGHEST)


def _row_spec(c):
    return pl.BlockSpec((R_BLK, c), lambda i: (i, 0))


def _full_spec(shape):
    return pl.BlockSpec(shape, lambda i: tuple(0 for _ in shape))


def _stats_spec(c):
    return pl.BlockSpec((8, c), lambda i: (0, 0))


def _tc_colstats(x):
    c = x.shape[1]

    def body(x_ref, s_ref, q_ref):
        _accum_stats(pl.program_id(0), x_ref[...], s_ref, q_ref)

    return pl.pallas_call(
        body,
        grid=(N_BLKS,),
        in_specs=[_row_spec(c)],
        out_specs=[_stats_spec(c), _stats_spec(c)],
        out_shape=[jax.ShapeDtypeStruct((8, c), jnp.float32)] * 2,
    )(x)


def _tc_make_y(x, s0, q0, g0, b0, deg):
    c = x.shape[1]

    def body(x_ref, s_ref, q_ref, g_ref, b_ref, d_ref, y_ref):
        scale, shift = _bn_coeffs(s_ref, q_ref, g_ref, b_ref)
        dinv = lax.rsqrt(d_ref[:, 0:1] + 1.0)
        y_ref[...] = dinv * (x_ref[...] * scale + shift)

    return pl.pallas_call(
        body,
        grid=(N_BLKS,),
        in_specs=[_row_spec(c), _stats_spec(c), _stats_spec(c),
                  _full_spec((1, c)), _full_spec((1, c)),
                  pl.BlockSpec((R_BLK, 1), lambda i: (i, 0))],
        out_specs=[_row_spec(c)],
        out_shape=[jax.ShapeDtypeStruct((N_NODES, c), jnp.float32)],
    )(x, s0, q0, g0, b0, deg)[0]


def _tc_layer1(agg1, y, deg, w1, b1):
    cin, cout = w1.shape

    def body(a_ref, y_ref, d_ref, w_ref, b_ref, r_ref, s_ref, q_ref):
        dinv = lax.rsqrt(d_ref[:, 0:1] + 1.0)
        u = dinv * (a_ref[...] + y_ref[...])
        r = jnp.maximum(_dot(u, w_ref[...]) + b_ref[...], 0.0)
        r_ref[...] = r
        _accum_stats(pl.program_id(0), r, s_ref, q_ref)

    return pl.pallas_call(
        body,
        grid=(N_BLKS,),
        in_specs=[_row_spec(cin), _row_spec(cin),
                  pl.BlockSpec((R_BLK, 1), lambda i: (i, 0)),
                  _full_spec((cin, cout)), _full_spec((1, cout))],
        out_specs=[_row_spec(cout), _stats_spec(cout), _stats_spec(cout)],
        out_shape=[jax.ShapeDtypeStruct((N_NODES, cout), jnp.float32),
                   jax.ShapeDtypeStruct((8, cout), jnp.float32),
                   jax.ShapeDtypeStruct((8, cout), jnp.float32)],
    )(agg1, y, deg, w1, b1)


def _tc_bn_apply(r, s, q, g, b):
    c = r.shape[1]

    def body(r_ref, s_ref, q_ref, g_ref, b_ref, o_ref):
        scale, shift = _bn_coeffs(s_ref, q_ref, g_ref, b_ref)
        o_ref[...] = r_ref[...] * scale + shift

    return pl.pallas_call(
        body,
        grid=(N_BLKS,),
        in_specs=[_row_spec(c), _stats_spec(c), _stats_spec(c),
                  _full_spec((1, c)), _full_spec((1, c))],
        out_specs=[_row_spec(c)],
        out_shape=[jax.ShapeDtypeStruct((N_NODES, c), jnp.float32)],
    )(r, s, q, g, b)[0]


def _tc_graphconv(agg, h, w_rel, w_root, b):
    cin, cout = w_rel.shape

    def body(a_ref, h_ref, wr_ref, wo_ref, b_ref, r_ref, s_ref, q_ref):
        t = _dot(a_ref[...], wr_ref[...]) + _dot(h_ref[...], wo_ref[...])
        r = jnp.maximum(t + b_ref[...], 0.0)
        r_ref[...] = r
        _accum_stats(pl.program_id(0), r, s_ref, q_ref)

    return pl.pallas_call(
        body,
        grid=(N_BLKS,),
        in_specs=[_row_spec(cin), _row_spec(cin),
                  _full_spec((cin, cout)), _full_spec((cin, cout)),
                  _full_spec((1, cout))],
        out_specs=[_row_spec(cout), _stats_spec(cout), _stats_spec(cout)],
        out_shape=[jax.ShapeDtypeStruct((N_NODES, cout), jnp.float32),
                   jax.ShapeDtypeStruct((8, cout), jnp.float32),
                   jax.ShapeDtypeStruct((8, cout), jnp.float32)],
    )(agg, h, w_rel, w_root, b)


def _tc_bn_matmul(r, s, q, g, b, w):
    c = r.shape[1]
    cout = w.shape[1]

    def body(r_ref, s_ref, q_ref, g_ref, b_ref, w_ref, h_ref, p_ref):
        scale, shift = _bn_coeffs(s_ref, q_ref, g_ref, b_ref)
        h = r_ref[...] * scale + shift
        h_ref[...] = h
        p_ref[...] = _dot(h, w_ref[...])

    return pl.pallas_call(
        body,
        grid=(N_BLKS,),
        in_specs=[_row_spec(c), _stats_spec(c), _stats_spec(c),
                  _full_spec((1, c)), _full_spec((1, c)),
                  _full_spec((c, cout))],
        out_specs=[_row_spec(c), _row_spec(cout)],
        out_shape=[jax.ShapeDtypeStruct((N_NODES, c), jnp.float32),
                   jax.ShapeDtypeStruct((N_NODES, cout), jnp.float32)],
    )(r, s, q, g, b, w)


def _tc_layer3_tail(agg3, h2, w_root, b3):
    cin, cout = w_root.shape

    def body(a_ref, h_ref, w_ref, b_ref, r_ref, s_ref, q_ref):
        t = a_ref[...] + _dot(h_ref[...], w_ref[...])
        r = jnp.maximum(t + b_ref[...], 0.0)
        r_ref[...] = r
        _accum_stats(pl.program_id(0), r, s_ref, q_ref)

    return pl.pallas_call(
        body,
        grid=(N_BLKS,),
        in_specs=[_row_spec(cout), _row_spec(cin),
                  _full_spec((cin, cout)), _full_spec((1, cout))],
        out_specs=[_row_spec(cout), _stats_spec(cout), _stats_spec(cout)],
        out_shape=[jax.ShapeDtypeStruct((N_NODES, cout), jnp.float32),
                   jax.ShapeDtypeStruct((8, cout), jnp.float32),
                   jax.ShapeDtypeStruct((8, cout), jnp.float32)],
    )(agg3, h2, w_root, b3)


def kernel(x, edge_index, gamma0, beta0, W1, b1, gamma1, beta1,
           W2_rel, b2, W2_root, gamma2, beta2,
           W3_rel, b3, W3_root, gamma3, beta3):
    ei = edge_index.astype(jnp.int32)
    g0, b0 = gamma0.reshape(1, -1), beta0.reshape(1, -1)
    g1, b1r = gamma1.reshape(1, -1), b1.reshape(1, -1)
    be1 = beta1.reshape(1, -1)
    g2, b2r, be2 = gamma2.reshape(1, -1), b2.reshape(1, -1), beta2.reshape(1, -1)
    g3, b3r, be3 = gamma3.reshape(1, -1), b3.reshape(1, -1), beta3.reshape(1, -1)

    elist, counts, degflat = _bucket(ei)
    deg = degflat[:N_NODES].reshape(N_NODES, 1)

    s0, q0 = _tc_colstats(x)
    y = _tc_make_y(x, s0, q0, g0, b0, deg)
    agg1 = _aggregate(y, elist, counts)
    r1, s1, q1 = _tc_layer1(agg1, y, deg, W1, b1r)

    h1 = _tc_bn_apply(r1, s1, q1, g1, be1)
    agg2 = jnp.concatenate(
        [_aggregate(h1[:, :AGG_C], elist, counts),
         _aggregate(h1[:, AGG_C:], elist, counts)], axis=1)
    r2, s2, q2 = _tc_graphconv(agg2, h1, W2_rel, W2_root, b2r)

    h2, p = _tc_bn_matmul(r2, s2, q2, g2, be2, W3_rel)
    agg3 = _aggregate(p, elist, counts)
    r3, s3, q3 = _tc_layer3_tail(agg3, h2, W3_root, b3r)

    return _tc_bn_apply(r3, s3, q3, g3, be3)
